# Initial kernel scaffold; baseline (speedup 1.0000x reference)
import jax
import jax.numpy as jnp
from jax.experimental import pallas as pl


def kernel(x, edge_index, class_edge_index, physical_edge_index, W_msg1, b_msg1, W_self1, b_self1, W_msg2, b_msg2, W_self2, b_self2, W_e1, W_e2, Wd_l, bd_l, Wd_r):
    raise NotImplementedError("write your pallas kernel here")


# speedup vs baseline: 1.6426x; 1.6426x over previous
import functools

import jax
import jax.numpy as jnp
from jax import lax
from jax.experimental import pallas as pl
from jax.experimental.pallas import tpu as pltpu
from jax.experimental.pallas import tpu_sc as plsc

N = 10000
E = 320000
D = 128

NC = 2
NS = 16
NW = NC * NS

EPW = E // NW
CH = 200
NCH = EPW // CH

ROWS_PER_SUB = N // NS


def _sage_scatter(x, src, dst, zeros_nd):
    mesh = plsc.VectorSubcoreMesh(core_axis_name="c", subcore_axis_name="s")

    @functools.partial(
        pl.kernel,
        out_type=jax.ShapeDtypeStruct((NC, N, D), jnp.float32),
        mesh=mesh,
        scratch_types=[
            pltpu.VMEM_SHARED((N, D), jnp.float32),
            pltpu.VMEM((CH,), jnp.int32),
            pltpu.VMEM((CH,), jnp.int32),
            pltpu.VMEM((CH, D), jnp.float32),
            pltpu.SemaphoreType.DMA,
        ],
    )
    def k(x_hbm, src_hbm, dst_hbm, znd_hbm, sums_out,
          acc_sp, sidx, didx, rows, sem):
        c = lax.axis_index("c")
        s = lax.axis_index("s")
        wid = s * NC + c

        @pl.when(s < 10)
        def _():
            pltpu.sync_copy(znd_hbm.at[pl.ds(s * 1000, 1000)],
                            acc_sp.at[pl.ds(s * 1000, 1000)])
        plsc.subcore_barrier()

        base = wid * EPW

        def body(ci, carry):
            off = base + ci * CH
            pltpu.sync_copy(src_hbm.at[pl.ds(off, CH)], sidx)
            pltpu.sync_copy(dst_hbm.at[pl.ds(off, CH)], didx)
            pltpu.async_copy(x_hbm.at[sidx], rows, sem).wait()
            pltpu.sync_copy(rows, acc_sp.at[didx], add=True)
            return carry

        lax.fori_loop(0, NCH, body, 0)
        plsc.subcore_barrier()

        @pl.when(s < 10)
        def _():
            pltpu.sync_copy(acc_sp.at[pl.ds(s * 1000, 1000)],
                            sums_out.at[c].at[pl.ds(s * 1000, 1000)])

    return k(x, src, dst, zeros_nd)


CHC = 1000
NP = N + 240


def _count_hist(dst):
    mesh = plsc.VectorSubcoreMesh(core_axis_name="c", subcore_axis_name="s")

    @functools.partial(
        pl.kernel,
        out_type=jax.ShapeDtypeStruct((NW, 1, NP), jnp.int32),
        mesh=mesh,
        scratch_types=[
            pltpu.VMEM((CHC + 16,), jnp.int32),
            pltpu.VMEM((1, NP), jnp.int32),
        ],
    )
    def k(dst_hbm, cnt_out, dbuf, hist):
        c = lax.axis_index("c")
        s = lax.axis_index("s")
        wid = s * NC + c
        lanes = _iota16()
        oh = jnp.where(lanes == 0, 1, 0).astype(jnp.int32)

        def z(i, carry):
            hist[0, pl.ds(i * 16, 16)] = jnp.zeros((16,), jnp.int32)
            return carry

        lax.fori_loop(0, NP // 16, z, 0)
        base = wid * EPW

        def body(ci, carry):
            pltpu.sync_copy(dst_hbm.at[pl.ds(base + ci * CHC, CHC)],
                            dbuf.at[pl.ds(0, CHC)])

            def grp(v, carry2):
                dv = dbuf[pl.ds(v * 16, 16)]
                for l in range(16):
                    d = dv[l]
                    hist[0, pl.ds(d, 16)] = hist[0, pl.ds(d, 16)] + oh
                return carry2

            lax.fori_loop(0, CHC // 16, grp, 0)
            return carry

        lax.fori_loop(0, EPW // CHC, body, 0)
        pltpu.sync_copy(hist, cnt_out.at[wid])

    return k(dst)


def _layer_dense(x, sums, cnts, W_msg, b_msg, W_self, b_self):
    BM = 1000

    def body(x_ref, s_ref, c_ref, wm_ref, bm_ref, ws_ref, bs_ref, o_ref):
        ssum = s_ref[0] + s_ref[1]
        cnt = c_ref[...]
        agg = ssum / jnp.maximum(cnt, 1.0)
        h = jnp.dot(agg, wm_ref[...], preferred_element_type=jnp.float32)
        h = h + jnp.dot(x_ref[...], ws_ref[...],
                        preferred_element_type=jnp.float32)
        o_ref[...] = jnp.maximum(h + bm_ref[...] + bs_ref[...], 0.0)

    return pl.pallas_call(
        body,
        grid=(N // BM,),
        in_specs=[
            pl.BlockSpec((BM, D), lambda i: (i, 0)),
            pl.BlockSpec((NC, BM, D), lambda i: (0, i, 0)),
            pl.BlockSpec((BM, 1), lambda i: (i, 0)),
            pl.BlockSpec((D, D), lambda i: (0, 0)),
            pl.BlockSpec((1, D), lambda i: (0, 0)),
            pl.BlockSpec((D, D), lambda i: (0, 0)),
            pl.BlockSpec((1, D), lambda i: (0, 0)),
        ],
        out_specs=pl.BlockSpec((BM, D), lambda i: (i, 0)),
        out_shape=jax.ShapeDtypeStruct((N, D), jnp.float32),
    )(x, sums, cnts, W_msg, b_msg.reshape(1, D),
      W_self, b_self.reshape(1, D))


def _encoder_proj(h, W_e1, W_e2, Wd_r):
    BM = 1000

    def body(h_ref, w1_ref, w2_ref, wr_ref, p_ref, q_ref, r1_ref, r2_ref):
        p = jnp.dot(h_ref[...], w1_ref[...], preferred_element_type=jnp.float32)
        q = jnp.dot(h_ref[...], w2_ref[...], preferred_element_type=jnp.float32)
        p_ref[...] = p
        q_ref[...] = q
        r1_ref[...] = jnp.dot(p, wr_ref[...], preferred_element_type=jnp.float32)
        r2_ref[...] = jnp.dot(q, wr_ref[...], preferred_element_type=jnp.float32)

    return pl.pallas_call(
        body,
        grid=(N // BM,),
        in_specs=[
            pl.BlockSpec((BM, D), lambda i: (i, 0)),
            pl.BlockSpec((D, D), lambda i: (0, 0)),
            pl.BlockSpec((D, D), lambda i: (0, 0)),
            pl.BlockSpec((D, 1), lambda i: (0, 0)),
        ],
        out_specs=[
            pl.BlockSpec((BM, D), lambda i: (i, 0)),
            pl.BlockSpec((BM, D), lambda i: (i, 0)),
            pl.BlockSpec((BM, 1), lambda i: (i, 0)),
            pl.BlockSpec((BM, 1), lambda i: (i, 0)),
        ],
        out_shape=[
            jax.ShapeDtypeStruct((N, D), jnp.float32),
            jax.ShapeDtypeStruct((N, D), jnp.float32),
            jax.ShapeDtypeStruct((N, 1), jnp.float32),
            jax.ShapeDtypeStruct((N, 1), jnp.float32),
        ],
    )(h, W_e1, W_e2, Wd_r)


NB = 625
SB = 512
NBP = 640
HP = 656
CH3 = 400
CK = 128
SE = NB * SB + NB * 128 + 128


def _iota16():
    return lax.broadcasted_iota(jnp.int32, (16,), 0)


def _bucket_hist(pdst):
    mesh = plsc.VectorSubcoreMesh(core_axis_name="c", subcore_axis_name="s")

    @functools.partial(
        pl.kernel,
        out_type=jax.ShapeDtypeStruct((NW, 1, HP), jnp.int32),
        mesh=mesh,
        scratch_types=[
            pltpu.VMEM((CH3 + 16,), jnp.int32),
            pltpu.VMEM((1, HP), jnp.int32),
        ],
    )
    def k(pdst_hbm, hist_out, pbuf, hist):
        c = lax.axis_index("c")
        s = lax.axis_index("s")
        wid = s * NC + c
        lanes = _iota16()
        oh = jnp.where(lanes == 0, 1, 0).astype(jnp.int32)

        def z(i, carry):
            hist[0, pl.ds(i * 16, 16)] = jnp.zeros((16,), jnp.int32)
            return carry

        lax.fori_loop(0, HP // 16, z, 0)
        base = wid * EPW

        def body(ci, carry):
            pltpu.sync_copy(pdst_hbm.at[pl.ds(base + ci * CH3, CH3)],
                            pbuf.at[pl.ds(0, CH3)])

            def grp(v, carry2):
                bv = lax.shift_right_logical(pbuf[pl.ds(v * 16, 16)], 9)
                for l in range(16):
                    b = bv[l]
                    hist[0, pl.ds(b, 16)] = hist[0, pl.ds(b, 16)] + oh
                return carry2

            lax.fori_loop(0, CH3 // 16, grp, 0)
            return carry

        lax.fori_loop(0, EPW // CH3, body, 0)
        pltpu.sync_copy(hist, hist_out.at[wid])

    return k(pdst)


def _bucket_rank(hist, psrc, pdst, esrc, edst):
    mesh = plsc.VectorSubcoreMesh(core_axis_name="c", subcore_axis_name="s")

    @functools.partial(
        pl.kernel,
        out_type=(jax.ShapeDtypeStruct((SE,), jnp.int32),
                  jax.ShapeDtypeStruct((SE,), jnp.int32),
                  jax.ShapeDtypeStruct((SE,), jnp.int32),
                  jax.ShapeDtypeStruct((NBP,), jnp.int32),
                  jax.ShapeDtypeStruct((NBP,), jnp.int32)),
        mesh=mesh,
        scratch_types=[
            pltpu.VMEM((NW, 1, HP), jnp.int32),
            pltpu.VMEM((HP,), jnp.int32),
            pltpu.VMEM((HP,), jnp.int32),
            pltpu.VMEM((HP,), jnp.int32),
            pltpu.SMEM((HP,), jnp.int32),
            pltpu.SMEM((HP,), jnp.int32),
            pltpu.VMEM((CH3,), jnp.int32),
            pltpu.VMEM((CH3,), jnp.int32),
            pltpu.VMEM((CH3,), jnp.int32),
            pltpu.VMEM((CH3,), jnp.int32),
            pltpu.VMEM((CH3,), jnp.int32),
            pltpu.VMEM((CH3,), jnp.int32),
            pltpu.SemaphoreType.DMA,
        ],
    )
    def k(hist_hbm, psrc_hbm, pdst_hbm, esrc_hbm, edst_hbm,
          s2_out, d2_out, li_out, tot_out, base_out,
          histv, tot, par, basev, bases, ctr, bbuf, psbuf, s2b, d2b, lib,
          posb, sem):
        c = lax.axis_index("c")
        s = lax.axis_index("s")
        wid = s * NC + c
        lanes = _iota16()
        pltpu.sync_copy(hist_hbm, histv)

        for j in range(HP // 16):
            tot[pl.ds(j * 16, 16)] = jnp.zeros((16,), jnp.int32)
            par[pl.ds(j * 16, 16)] = jnp.zeros((16,), jnp.int32)

        def accw(w, carry):
            use = (w < wid).astype(jnp.int32)
            for j in range(HP // 16):
                row = histv[w, 0, pl.ds(j * 16, 16)]
                tot[pl.ds(j * 16, 16)] = tot[pl.ds(j * 16, 16)] + row
                par[pl.ds(j * 16, 16)] = (par[pl.ds(j * 16, 16)]
                                          + row * use)
            return carry

        lax.fori_loop(0, NW, accw, 0)

        def cum(b, acc):
            bases[b] = acc
            ctr[b] = acc + par[pl.ds(b, 16)][0]
            nxt = acc + tot[pl.ds(b, 16)][0]
            return (nxt + 127) & jnp.int32(~127)

        lax.fori_loop(0, NB, cum, jnp.int32(0))

        def bv(j, carry):
            vec = jnp.zeros((16,), jnp.int32)
            for l in range(16):
                vec = jnp.where(lanes == l, bases[j * 16 + l], vec)
            basev[pl.ds(j * 16, 16)] = vec
            return carry

        lax.fori_loop(0, NBP // 16, bv, 0)

        @pl.when(wid == 0)
        def _():
            pltpu.sync_copy(tot.at[pl.ds(0, NBP)], tot_out)
            pltpu.sync_copy(basev.at[pl.ds(0, NBP)], base_out)

        base_e = wid * EPW

        def body(ci, carry):
            off = base_e + ci * CH3
            pltpu.sync_copy(pdst_hbm.at[pl.ds(off, CH3)], bbuf)
            pltpu.sync_copy(psrc_hbm.at[pl.ds(off, CH3)], psbuf)
            cp1 = pltpu.async_copy(esrc_hbm.at[psbuf], s2b, sem)
            cp1.wait()
            cp2 = pltpu.async_copy(edst_hbm.at[psbuf], d2b, sem)
            cp2.wait()
            for v in range(CH3 // 16):
                pd = bbuf[pl.ds(v * 16, 16)]
                lib[pl.ds(v * 16, 16)] = pd & (SB - 1)
                bbuf[pl.ds(v * 16, 16)] = lax.shift_right_logical(pd, 9)

            def rank(v, carry2):
                bvv = bbuf[pl.ds(v * 16, 16)]
                posvec = jnp.zeros((16,), jnp.int32)
                for l in range(16):
                    b = bvv[l]
                    pos = ctr[b]
                    ctr[b] = pos + 1
                    posvec = jnp.where(lanes == l, pos, posvec)
                posb[pl.ds(v * 16, 16)] = posvec
                return carry2

            lax.fori_loop(0, CH3 // 16, rank, 0)
            pltpu.sync_copy(s2b, s2_out.at[posb])
            pltpu.sync_copy(d2b, d2_out.at[posb])
            pltpu.sync_copy(lib, li_out.at[posb])
            return carry

        lax.fori_loop(0, EPW // CH3, body, 0)

    return k(hist, psrc, pdst, esrc, edst)


def _bucket_max(s2s, d2s, lis, tot, basep, P, Q, r1, r2, esrc, edst):
    mesh = plsc.VectorSubcoreMesh(core_axis_name="c", subcore_axis_name="s")

    @functools.partial(
        pl.kernel,
        out_type=(jax.ShapeDtypeStruct((E, D), jnp.float32),
                  jax.ShapeDtypeStruct((E,), jnp.int32),
                  jax.ShapeDtypeStruct((E,), jnp.float32)),
        mesh=mesh,
        scratch_types=[
            pltpu.VMEM((SB, D), jnp.float32),
            pltpu.VMEM((SB + 16,), jnp.int32),
            pltpu.VMEM((HP,), jnp.int32),
            pltpu.VMEM((HP,), jnp.int32),
            pltpu.VMEM((CK + 16,), jnp.int32),
            pltpu.VMEM((CK + 16,), jnp.int32),
            pltpu.VMEM((CK + 16,), jnp.int32),
            pltpu.VMEM((CK, D), jnp.float32),
            pltpu.VMEM((CK, D), jnp.float32),
            pltpu.VMEM((SB,), jnp.int32),
            pltpu.VMEM((SB,), jnp.int32),
            pltpu.VMEM((SB,), jnp.float32),
            pltpu.VMEM((SB,), jnp.float32),
            pltpu.VMEM((SB,), jnp.float32),
            pltpu.SemaphoreType.DMA,
            pltpu.SemaphoreType.DMA,
        ],
    )
    def k(s2_hbm, d2_hbm, li_hbm, tot_hbm, base_hbm, p_hbm, q_hbm,
          r1_hbm, r2_hbm, esrc_hbm, edst_hbm,
          agg_out, tch_out, par_out,
          acc, mrow, totv, basev, s2b, d2b, lib, prows, qrows,
          peb, pdb, pr1, pr2, pw, sem1, sem2):
        c = lax.axis_index("c")
        s = lax.axis_index("s")
        wid = s * NC + c
        pltpu.sync_copy(tot_hbm, totv.at[pl.ds(0, NBP)])
        pltpu.sync_copy(base_hbm, basev.at[pl.ds(0, NBP)])
        lanes = _iota16()

        lo = (NB * wid + NW - 1) // NW
        hi = (NB * (wid + 1) + NW - 1) // NW

        def bucket(b, carry):
            n = totv[pl.ds(b, 16)][0]
            b0 = basev[pl.ds(b, 16)][0]

            def zm(i, carry2):
                mrow[pl.ds(i * 16, 16)] = jnp.zeros((16,), jnp.int32)
                return carry2

            lax.fori_loop(0, (SB + 16) // 16, zm, 0)
            nch = (n + CK - 1) // CK

            def chunk(g, carry2):
                off = pl.multiple_of(b0 + g * CK, 128)
                pltpu.sync_copy(s2_hbm.at[pl.ds(off, CK)],
                                s2b.at[pl.ds(0, CK)])
                pltpu.sync_copy(d2_hbm.at[pl.ds(off, CK)],
                                d2b.at[pl.ds(0, CK)])
                pltpu.sync_copy(li_hbm.at[pl.ds(off, CK)],
                                lib.at[pl.ds(0, CK)])
                rem = n - g * CK
                for v in range(CK // 16):
                    lid = lanes + v * 16
                    m = lid < rem
                    zero = jnp.zeros((16,), jnp.int32)
                    s2v = jnp.where(m, s2b[pl.ds(v * 16, 16)], zero)
                    d2v = jnp.where(m, d2b[pl.ds(v * 16, 16)], zero)
                    s2b[pl.ds(v * 16, 16)] = s2v
                    d2b[pl.ds(v * 16, 16)] = d2v
                cp1 = pltpu.async_copy(p_hbm.at[s2b.at[pl.ds(0, CK)]],
                                       prows, sem1)
                cp2 = pltpu.async_copy(q_hbm.at[d2b.at[pl.ds(0, CK)]],
                                       qrows, sem2)
                cp1.wait()
                cp2.wait()
                cnt = jnp.minimum(rem, CK)

                def edge(j, carry3):
                    li = lib[pl.ds(j, 16)][0]
                    t = mrow[pl.ds(li, 16)][0]

                    @pl.when(t == 1)
                    def _():
                        for kk in range(D // 16):
                            sl = pl.ds(kk * 16, 16)
                            row = prows[j, sl] + qrows[j, sl]
                            acc[li, sl] = jnp.maximum(acc[li, sl], row)

                    @pl.when(t == 0)
                    def _():
                        for kk in range(D // 16):
                            sl = pl.ds(kk * 16, 16)
                            acc[li, sl] = prows[j, sl] + qrows[j, sl]
                        mv = mrow[pl.ds(li, 16)]
                        mrow[pl.ds(li, 16)] = jnp.where(lanes == 0, 1, mv)

                    return carry3

                lax.fori_loop(0, cnt, edge, 0)
                return carry2

            lax.fori_loop(0, nch, chunk, 0)

            orow = pl.multiple_of(b * SB, 128)
            pltpu.sync_copy(acc, agg_out.at[pl.ds(orow, SB)])
            pltpu.sync_copy(mrow.at[pl.ds(0, SB)],
                            tch_out.at[pl.ds(orow, SB)])

            pltpu.sync_copy(esrc_hbm.at[pl.ds(orow, SB)], peb)
            pltpu.sync_copy(edst_hbm.at[pl.ds(orow, SB)], pdb)
            cp1 = pltpu.async_copy(r1_hbm.at[peb], pr1, sem1)
            cp2 = pltpu.async_copy(r2_hbm.at[pdb], pr2, sem2)
            cp1.wait()
            cp2.wait()
            for v in range(SB // 16):
                sl = pl.ds(v * 16, 16)
                pw[sl] = pr1[sl] + pr2[sl]
            pltpu.sync_copy(pw, par_out.at[pl.ds(orow, SB)])
            return carry

        lax.fori_loop(lo, hi, bucket, 0)

    return k(s2s, d2s, lis, tot, basep, P, Q, r1, r2, esrc, edst)


def _decoder_epilogue(agg, touched, partial, Wd_l, bd_l):
    BM = 2000

    def body(a_ref, t_ref, p_ref, wl_ref, bl_ref, o_ref):
        a = jnp.where(t_ref[...] > 0, a_ref[...], 0.0)
        o_ref[...] = (jnp.dot(a, wl_ref[...],
                              preferred_element_type=jnp.float32)
                      + bl_ref[...] + p_ref[...])

    return pl.pallas_call(
        body,
        grid=(E // BM,),
        in_specs=[
            pl.BlockSpec((BM, D), lambda i: (i, 0)),
            pl.BlockSpec((BM, 1), lambda i: (i, 0)),
            pl.BlockSpec((BM, 1), lambda i: (i, 0)),
            pl.BlockSpec((D, 1), lambda i: (0, 0)),
            pl.BlockSpec((1, 1), lambda i: (0, 0)),
        ],
        out_specs=pl.BlockSpec((BM, 1), lambda i: (i, 0)),
        out_shape=jax.ShapeDtypeStruct((E, 1), jnp.float32),
    )(agg, touched.reshape(E, 1), partial.reshape(E, 1), Wd_l,
      bd_l.reshape(1, 1))


def kernel(x, edge_index, class_edge_index, physical_edge_index,
           W_msg1, b_msg1, W_self1, b_self1,
           W_msg2, b_msg2, W_self2, b_self2,
           W_e1, W_e2, Wd_l, bd_l, Wd_r):
    src = edge_index[0]
    dst = edge_index[1]
    zeros_nd = jnp.zeros((N, D), jnp.float32)

    cnt_raw = _count_hist(dst)
    cnts = jnp.sum(cnt_raw[:, 0, :N], axis=0).astype(jnp.float32)
    cnts = cnts.reshape(N, 1)

    sums1 = _sage_scatter(x, src, dst, zeros_nd)
    h1 = _layer_dense(x, sums1, cnts, W_msg1, b_msg1, W_self1, b_self1)
    sums2 = _sage_scatter(h1, src, dst, zeros_nd)
    h2 = _layer_dense(h1, sums2, cnts, W_msg2, b_msg2, W_self2, b_self2)

    P, Q, r1, r2 = _encoder_proj(h2, W_e1, W_e2, Wd_r)

    psrc = physical_edge_index[0]
    pdst = physical_edge_index[1]
    hist = _bucket_hist(pdst)
    s2s, d2s, lis, tot, basep = _bucket_rank(hist, psrc, pdst, src, dst)
    agg, touched, partial = _bucket_max(s2s, d2s, lis, tot, basep,
                                        P, Q, r1.reshape(N), r2.reshape(N),
                                        src, dst)
    return _decoder_epilogue(agg, touched, partial, Wd_l, bd_l)


# speedup vs baseline: 1.7749x; 1.0806x over previous
import functools

import jax
import jax.numpy as jnp
from jax import lax
from jax.experimental import pallas as pl
from jax.experimental.pallas import tpu as pltpu
from jax.experimental.pallas import tpu_sc as plsc

N = 10000
E = 320000
D = 128

NC = 2
NS = 16
NW = NC * NS

EPW = E // NW
CH = 200
NCH = EPW // CH

ROWS_PER_SUB = N // NS


def _sage_scatter(x, src, dst, zeros_nd):
    mesh = plsc.VectorSubcoreMesh(core_axis_name="c", subcore_axis_name="s")

    @functools.partial(
        pl.kernel,
        out_type=jax.ShapeDtypeStruct((NC, N, D), jnp.float32),
        mesh=mesh,
        scratch_types=[
            pltpu.VMEM_SHARED((N, D), jnp.float32),
            pltpu.VMEM((CH,), jnp.int32),
            pltpu.VMEM((CH,), jnp.int32),
            pltpu.VMEM((CH, D), jnp.float32),
            pltpu.SemaphoreType.DMA,
        ],
    )
    def k(x_hbm, src_hbm, dst_hbm, znd_hbm, sums_out,
          acc_sp, sidx, didx, rows, sem):
        c = lax.axis_index("c")
        s = lax.axis_index("s")
        wid = s * NC + c

        @pl.when(s < 10)
        def _():
            pltpu.sync_copy(znd_hbm.at[pl.ds(s * 1000, 1000)],
                            acc_sp.at[pl.ds(s * 1000, 1000)])
        plsc.subcore_barrier()

        base = wid * EPW

        def body(ci, carry):
            off = base + ci * CH
            pltpu.sync_copy(src_hbm.at[pl.ds(off, CH)], sidx)
            pltpu.sync_copy(dst_hbm.at[pl.ds(off, CH)], didx)
            pltpu.async_copy(x_hbm.at[sidx], rows, sem).wait()
            pltpu.sync_copy(rows, acc_sp.at[didx], add=True)
            return carry

        lax.fori_loop(0, NCH, body, 0)
        plsc.subcore_barrier()

        @pl.when(s < 10)
        def _():
            pltpu.sync_copy(acc_sp.at[pl.ds(s * 1000, 1000)],
                            sums_out.at[c].at[pl.ds(s * 1000, 1000)])

    return k(x, src, dst, zeros_nd)


CHC = 1000
NP = N + 240


def _count_hist(dst):
    mesh = plsc.VectorSubcoreMesh(core_axis_name="c", subcore_axis_name="s")

    @functools.partial(
        pl.kernel,
        out_type=jax.ShapeDtypeStruct((NW, 1, NP), jnp.int32),
        mesh=mesh,
        scratch_types=[
            pltpu.VMEM((CHC + 16,), jnp.int32),
            pltpu.VMEM((1, NP), jnp.int32),
        ],
    )
    def k(dst_hbm, cnt_out, dbuf, hist):
        c = lax.axis_index("c")
        s = lax.axis_index("s")
        wid = s * NC + c
        lanes = _iota16()
        oh = jnp.where(lanes == 0, 1, 0).astype(jnp.int32)

        def z(i, carry):
            hist[0, pl.ds(i * 16, 16)] = jnp.zeros((16,), jnp.int32)
            return carry

        lax.fori_loop(0, NP // 16, z, 0)
        base = wid * EPW

        def body(ci, carry):
            pltpu.sync_copy(dst_hbm.at[pl.ds(base + ci * CHC, CHC)],
                            dbuf.at[pl.ds(0, CHC)])

            def grp(v, carry2):
                dv = dbuf[pl.ds(v * 16, 16)]
                for l in range(16):
                    d = dv[l]
                    hist[0, pl.ds(d, 16)] = hist[0, pl.ds(d, 16)] + oh
                return carry2

            lax.fori_loop(0, CHC // 16, grp, 0)
            return carry

        lax.fori_loop(0, EPW // CHC, body, 0)
        pltpu.sync_copy(hist, cnt_out.at[wid])

    return k(dst)


def _layer_dense(x, sums, cnts, W_msg, b_msg, W_self, b_self):
    BM = 1000

    def body(x_ref, s_ref, c_ref, wm_ref, bm_ref, ws_ref, bs_ref, o_ref):
        ssum = s_ref[0] + s_ref[1]
        cnt = c_ref[...]
        agg = ssum / jnp.maximum(cnt, 1.0)
        h = jnp.dot(agg, wm_ref[...], preferred_element_type=jnp.float32)
        h = h + jnp.dot(x_ref[...], ws_ref[...],
                        preferred_element_type=jnp.float32)
        o_ref[...] = jnp.maximum(h + bm_ref[...] + bs_ref[...], 0.0)

    return pl.pallas_call(
        body,
        grid=(N // BM,),
        in_specs=[
            pl.BlockSpec((BM, D), lambda i: (i, 0)),
            pl.BlockSpec((NC, BM, D), lambda i: (0, i, 0)),
            pl.BlockSpec((BM, 1), lambda i: (i, 0)),
            pl.BlockSpec((D, D), lambda i: (0, 0)),
            pl.BlockSpec((1, D), lambda i: (0, 0)),
            pl.BlockSpec((D, D), lambda i: (0, 0)),
            pl.BlockSpec((1, D), lambda i: (0, 0)),
        ],
        out_specs=pl.BlockSpec((BM, D), lambda i: (i, 0)),
        out_shape=jax.ShapeDtypeStruct((N, D), jnp.float32),
    )(x, sums, cnts, W_msg, b_msg.reshape(1, D),
      W_self, b_self.reshape(1, D))


def _encoder_proj(h, W_e1, W_e2, Wd_r):
    BM = 1000

    def body(h_ref, w1_ref, w2_ref, wr_ref, p_ref, q_ref, r1_ref, r2_ref):
        p = jnp.dot(h_ref[...], w1_ref[...], preferred_element_type=jnp.float32)
        q = jnp.dot(h_ref[...], w2_ref[...], preferred_element_type=jnp.float32)
        p_ref[...] = p
        q_ref[...] = q
        r1_ref[...] = jnp.dot(p, wr_ref[...], preferred_element_type=jnp.float32)
        r2_ref[...] = jnp.dot(q, wr_ref[...], preferred_element_type=jnp.float32)

    return pl.pallas_call(
        body,
        grid=(N // BM,),
        in_specs=[
            pl.BlockSpec((BM, D), lambda i: (i, 0)),
            pl.BlockSpec((D, D), lambda i: (0, 0)),
            pl.BlockSpec((D, D), lambda i: (0, 0)),
            pl.BlockSpec((D, 1), lambda i: (0, 0)),
        ],
        out_specs=[
            pl.BlockSpec((BM, D), lambda i: (i, 0)),
            pl.BlockSpec((BM, D), lambda i: (i, 0)),
            pl.BlockSpec((BM, 1), lambda i: (i, 0)),
            pl.BlockSpec((BM, 1), lambda i: (i, 0)),
        ],
        out_shape=[
            jax.ShapeDtypeStruct((N, D), jnp.float32),
            jax.ShapeDtypeStruct((N, D), jnp.float32),
            jax.ShapeDtypeStruct((N, 1), jnp.float32),
            jax.ShapeDtypeStruct((N, 1), jnp.float32),
        ],
    )(h, W_e1, W_e2, Wd_r)


NB = 625
SB = 512
NBP = 640
HP = 656
CH3 = 2000
CK = 192
NEG = -1.0e38
SE = NB * SB + NB * 128 + 128


def _iota16():
    return lax.broadcasted_iota(jnp.int32, (16,), 0)


def _bucket_hist(pdst):
    mesh = plsc.VectorSubcoreMesh(core_axis_name="c", subcore_axis_name="s")

    @functools.partial(
        pl.kernel,
        out_type=jax.ShapeDtypeStruct((NW, 1, HP), jnp.int32),
        mesh=mesh,
        scratch_types=[
            pltpu.VMEM((CH3 + 16,), jnp.int32),
            pltpu.VMEM((1, HP), jnp.int32),
        ],
    )
    def k(pdst_hbm, hist_out, pbuf, hist):
        c = lax.axis_index("c")
        s = lax.axis_index("s")
        wid = s * NC + c
        lanes = _iota16()
        oh = jnp.where(lanes == 0, 1, 0).astype(jnp.int32)

        def z(i, carry):
            hist[0, pl.ds(i * 16, 16)] = jnp.zeros((16,), jnp.int32)
            return carry

        lax.fori_loop(0, HP // 16, z, 0)
        base = wid * EPW

        def body(ci, carry):
            pltpu.sync_copy(pdst_hbm.at[pl.ds(base + ci * CH3, CH3)],
                            pbuf.at[pl.ds(0, CH3)])

            def grp(v, carry2):
                bv = lax.shift_right_logical(pbuf[pl.ds(v * 16, 16)], 9)
                for l in range(16):
                    b = bv[l]
                    hist[0, pl.ds(b, 16)] = hist[0, pl.ds(b, 16)] + oh
                return carry2

            lax.fori_loop(0, CH3 // 16, grp, 0)
            return carry

        lax.fori_loop(0, EPW // CH3, body, 0)
        pltpu.sync_copy(hist, hist_out.at[wid])

    return k(pdst)


def _bucket_rank(hist, psrc, pdst, esrc, edst):
    mesh = plsc.VectorSubcoreMesh(core_axis_name="c", subcore_axis_name="s")

    @functools.partial(
        pl.kernel,
        out_type=(jax.ShapeDtypeStruct((SE,), jnp.int32),
                  jax.ShapeDtypeStruct((SE,), jnp.int32),
                  jax.ShapeDtypeStruct((SE,), jnp.int32),
                  jax.ShapeDtypeStruct((NBP,), jnp.int32),
                  jax.ShapeDtypeStruct((NBP,), jnp.int32)),
        mesh=mesh,
        scratch_types=[
            pltpu.VMEM((NW, 1, HP), jnp.int32),
            pltpu.VMEM((HP,), jnp.int32),
            pltpu.VMEM((HP,), jnp.int32),
            pltpu.VMEM((HP,), jnp.int32),
            pltpu.SMEM((HP,), jnp.int32),
            pltpu.SMEM((HP,), jnp.int32),
            pltpu.VMEM((CH3,), jnp.int32),
            pltpu.VMEM((CH3,), jnp.int32),
            pltpu.VMEM((CH3,), jnp.int32),
            pltpu.VMEM((CH3,), jnp.int32),
            pltpu.VMEM((CH3,), jnp.int32),
            pltpu.VMEM((CH3,), jnp.int32),
            pltpu.SemaphoreType.DMA,
            pltpu.SemaphoreType.DMA,
            pltpu.SemaphoreType.DMA,
        ],
    )
    def k(hist_hbm, psrc_hbm, pdst_hbm, esrc_hbm, edst_hbm,
          s2_out, d2_out, li_out, tot_out, base_out,
          histv, tot, par, basev, bases, ctr, bbuf, psbuf, s2b, d2b, lib,
          posb, sem, sem2, sem3):
        c = lax.axis_index("c")
        s = lax.axis_index("s")
        wid = s * NC + c
        lanes = _iota16()
        pltpu.sync_copy(hist_hbm, histv)

        for j in range(HP // 16):
            tot[pl.ds(j * 16, 16)] = jnp.zeros((16,), jnp.int32)
            par[pl.ds(j * 16, 16)] = jnp.zeros((16,), jnp.int32)

        def accw(w, carry):
            use = (w < wid).astype(jnp.int32)
            for j in range(HP // 16):
                row = histv[w, 0, pl.ds(j * 16, 16)]
                tot[pl.ds(j * 16, 16)] = tot[pl.ds(j * 16, 16)] + row
                par[pl.ds(j * 16, 16)] = (par[pl.ds(j * 16, 16)]
                                          + row * use)
            return carry

        lax.fori_loop(0, NW, accw, 0)

        def cum(b, acc):
            bases[b] = acc
            ctr[b] = acc + par[pl.ds(b, 16)][0]
            nxt = acc + tot[pl.ds(b, 16)][0]
            return (nxt + 127) & jnp.int32(~127)

        lax.fori_loop(0, NB, cum, jnp.int32(0))

        def bv(j, carry):
            vec = jnp.zeros((16,), jnp.int32)
            for l in range(16):
                vec = jnp.where(lanes == l, bases[j * 16 + l], vec)
            basev[pl.ds(j * 16, 16)] = vec
            return carry

        lax.fori_loop(0, NBP // 16, bv, 0)

        @pl.when(wid == 0)
        def _():
            pltpu.sync_copy(tot.at[pl.ds(0, NBP)], tot_out)
            pltpu.sync_copy(basev.at[pl.ds(0, NBP)], base_out)

        base_e = wid * EPW

        def body(ci, carry):
            off = base_e + ci * CH3
            cpa = pltpu.async_copy(pdst_hbm.at[pl.ds(off, CH3)], bbuf, sem)
            cpb = pltpu.async_copy(psrc_hbm.at[pl.ds(off, CH3)], psbuf, sem2)
            cpa.wait()
            cpb.wait()
            cp1 = pltpu.async_copy(esrc_hbm.at[psbuf], s2b, sem)
            cp2 = pltpu.async_copy(edst_hbm.at[psbuf], d2b, sem2)
            cp1.wait()
            cp2.wait()
            for v in range(CH3 // 16):
                pd = bbuf[pl.ds(v * 16, 16)]
                lib[pl.ds(v * 16, 16)] = pd & (SB - 1)
                bbuf[pl.ds(v * 16, 16)] = lax.shift_right_logical(pd, 9)

            def rank(v, carry2):
                bvv = bbuf[pl.ds(v * 16, 16)]
                posvec = jnp.zeros((16,), jnp.int32)
                for l in range(16):
                    b = bvv[l]
                    pos = ctr[b]
                    ctr[b] = pos + 1
                    posvec = jnp.where(lanes == l, pos, posvec)
                posb[pl.ds(v * 16, 16)] = posvec
                return carry2

            lax.fori_loop(0, CH3 // 16, rank, 0)
            cp3 = pltpu.async_copy(s2b, s2_out.at[posb], sem)
            cp4 = pltpu.async_copy(d2b, d2_out.at[posb], sem2)
            cp5 = pltpu.async_copy(lib, li_out.at[posb], sem3)
            cp3.wait()
            cp4.wait()
            cp5.wait()
            return carry

        lax.fori_loop(0, EPW // CH3, body, 0)

    return k(hist, psrc, pdst, esrc, edst)


def _bucket_max(s2s, d2s, lis, tot, basep, P, Q, r1, r2, esrc, edst):
    mesh = plsc.VectorSubcoreMesh(core_axis_name="c", subcore_axis_name="s")

    @functools.partial(
        pl.kernel,
        out_type=(jax.ShapeDtypeStruct((E, D), jnp.float32),
                  jax.ShapeDtypeStruct((E,), jnp.float32)),
        mesh=mesh,
        scratch_types=[
            pltpu.VMEM((SB + 1, D), jnp.float32),
            pltpu.VMEM((HP,), jnp.int32),
            pltpu.VMEM((HP,), jnp.int32),
            pltpu.VMEM((CK + 16,), jnp.int32),
            pltpu.VMEM((CK + 16,), jnp.int32),
            pltpu.VMEM((CK + 16,), jnp.int32),
            pltpu.VMEM((CK, D), jnp.float32),
            pltpu.VMEM((CK, D), jnp.float32),
            pltpu.VMEM((SB,), jnp.int32),
            pltpu.VMEM((SB,), jnp.int32),
            pltpu.VMEM((SB,), jnp.float32),
            pltpu.VMEM((SB,), jnp.float32),
            pltpu.VMEM((SB,), jnp.float32),
            pltpu.SemaphoreType.DMA,
            pltpu.SemaphoreType.DMA,
            pltpu.SemaphoreType.DMA,
        ],
    )
    def k(s2_hbm, d2_hbm, li_hbm, tot_hbm, base_hbm, p_hbm, q_hbm,
          r1_hbm, r2_hbm, esrc_hbm, edst_hbm,
          agg_out, par_out,
          acc, totv, basev, s2b, d2b, lib, prows, qrows,
          peb, pdb, pr1, pr2, pw, sem1, sem2, sem3):
        c = lax.axis_index("c")
        s = lax.axis_index("s")
        wid = s * NC + c
        pltpu.sync_copy(tot_hbm, totv.at[pl.ds(0, NBP)])
        pltpu.sync_copy(base_hbm, basev.at[pl.ds(0, NBP)])
        lanes = _iota16()
        neg16 = jnp.full((16,), NEG, jnp.float32)

        lo = (NB * wid + NW - 1) // NW
        hi = (NB * (wid + 1) + NW - 1) // NW

        def bucket(b, carry):
            n = totv[pl.ds(b, 16)][0]
            b0 = basev[pl.ds(b, 16)][0]

            def za(r, carry2):
                for kk in range(D // 16):
                    acc[r, pl.ds(kk * 16, 16)] = neg16
                return carry2

            lax.fori_loop(0, SB, za, 0)
            nch = (n + CK - 1) // CK

            def chunk(g, carry2):
                off = pl.multiple_of(b0 + g * CK, 8)
                cpa = pltpu.async_copy(s2_hbm.at[pl.ds(off, CK)],
                                       s2b.at[pl.ds(0, CK)], sem1)
                cpb = pltpu.async_copy(d2_hbm.at[pl.ds(off, CK)],
                                       d2b.at[pl.ds(0, CK)], sem2)
                cpc = pltpu.async_copy(li_hbm.at[pl.ds(off, CK)],
                                       lib.at[pl.ds(0, CK)], sem3)
                cpa.wait()
                cpb.wait()
                cpc.wait()
                rem = n - g * CK
                zero = jnp.zeros((16,), jnp.int32)
                trash = jnp.full((16,), SB, jnp.int32)
                for v in range(CK // 16):
                    m = (lanes + v * 16) < rem
                    sl = pl.ds(v * 16, 16)
                    s2b[sl] = jnp.where(m, s2b[sl], zero)
                    d2b[sl] = jnp.where(m, d2b[sl], zero)
                    lib[sl] = jnp.where(m, lib[sl], trash)
                cp1 = pltpu.async_copy(p_hbm.at[s2b.at[pl.ds(0, CK)]],
                                       prows, sem1)
                cp2 = pltpu.async_copy(q_hbm.at[d2b.at[pl.ds(0, CK)]],
                                       qrows, sem2)
                cp1.wait()
                cp2.wait()

                def grp(v, carry3):
                    livec = lib[pl.ds(v * 16, 16)]
                    for l in range(16):
                        li = livec[l]
                        j = v * 16 + l
                        for kk in range(D // 16):
                            sl = pl.ds(kk * 16, 16)
                            row = prows[j, sl] + qrows[j, sl]
                            acc[li, sl] = jnp.maximum(acc[li, sl], row)
                    return carry3

                lax.fori_loop(0, CK // 16, grp, 0)
                return carry2

            lax.fori_loop(0, nch, chunk, 0)

            orow = pl.multiple_of(b * SB, 128)
            pltpu.sync_copy(acc.at[pl.ds(0, SB)], agg_out.at[pl.ds(orow, SB)])

            cpa = pltpu.async_copy(esrc_hbm.at[pl.ds(orow, SB)], peb, sem1)
            cpb = pltpu.async_copy(edst_hbm.at[pl.ds(orow, SB)], pdb, sem2)
            cpa.wait()
            cpb.wait()
            cp1 = pltpu.async_copy(r1_hbm.at[peb], pr1, sem1)
            cp2 = pltpu.async_copy(r2_hbm.at[pdb], pr2, sem2)
            cp1.wait()
            cp2.wait()
            for v in range(SB // 16):
                sl = pl.ds(v * 16, 16)
                pw[sl] = pr1[sl] + pr2[sl]
            pltpu.sync_copy(pw, par_out.at[pl.ds(orow, SB)])
            return carry

        lax.fori_loop(lo, hi, bucket, 0)

    return k(s2s, d2s, lis, tot, basep, P, Q, r1, r2, esrc, edst)


def _decoder_epilogue(agg, partial, Wd_l, bd_l):
    BM = 2000

    def body(a_ref, p_ref, wl_ref, bl_ref, o_ref):
        a = a_ref[...]
        a = jnp.where(a > -1.0e37, a, 0.0)
        o_ref[...] = (jnp.dot(a, wl_ref[...],
                              preferred_element_type=jnp.float32)
                      + bl_ref[...] + p_ref[...])

    return pl.pallas_call(
        body,
        grid=(E // BM,),
        in_specs=[
            pl.BlockSpec((BM, D), lambda i: (i, 0)),
            pl.BlockSpec((BM, 1), lambda i: (i, 0)),
            pl.BlockSpec((D, 1), lambda i: (0, 0)),
            pl.BlockSpec((1, 1), lambda i: (0, 0)),
        ],
        out_specs=pl.BlockSpec((BM, 1), lambda i: (i, 0)),
        out_shape=jax.ShapeDtypeStruct((E, 1), jnp.float32),
    )(agg, partial.reshape(E, 1), Wd_l, bd_l.reshape(1, 1))


def kernel(x, edge_index, class_edge_index, physical_edge_index,
           W_msg1, b_msg1, W_self1, b_self1,
           W_msg2, b_msg2, W_self2, b_self2,
           W_e1, W_e2, Wd_l, bd_l, Wd_r):
    src = edge_index[0]
    dst = edge_index[1]
    zeros_nd = jnp.zeros((N, D), jnp.float32)

    cnt_raw = _count_hist(dst)
    cnts = jnp.sum(cnt_raw[:, 0, :N], axis=0).astype(jnp.float32)
    cnts = cnts.reshape(N, 1)

    sums1 = _sage_scatter(x, src, dst, zeros_nd)
    h1 = _layer_dense(x, sums1, cnts, W_msg1, b_msg1, W_self1, b_self1)
    sums2 = _sage_scatter(h1, src, dst, zeros_nd)
    h2 = _layer_dense(h1, sums2, cnts, W_msg2, b_msg2, W_self2, b_self2)

    P, Q, r1, r2 = _encoder_proj(h2, W_e1, W_e2, Wd_r)

    psrc = physical_edge_index[0]
    pdst = physical_edge_index[1]
    hist = _bucket_hist(pdst)
    s2s, d2s, lis, tot, basep = _bucket_rank(hist, psrc, pdst, src, dst)
    agg, partial = _bucket_max(s2s, d2s, lis, tot, basep,
                               P, Q, r1.reshape(N), r2.reshape(N),
                               src, dst)
    return _decoder_epilogue(agg, partial, Wd_l, bd_l)
